# Initial kernel scaffold; baseline (speedup 1.0000x reference)
import jax
import jax.numpy as jnp
from jax.experimental import pallas as pl


def kernel(input_ids, embed_tokens, lm_head_w):
    raise NotImplementedError("write your pallas kernel here")


# speedup vs baseline: 2.1798x; 2.1798x over previous
import functools

import jax
import jax.numpy as jnp
import numpy as np
from jax import lax
from jax.experimental import pallas as pl
from jax.experimental.pallas import tpu as pltpu
from jax.experimental.pallas import tpu_sc as plsc

_VOCAB = 20
_D = 8
_LANES = 16


def _table_body(e_ref, w_ref, t_ref):
    t_ref[...] = lax.dot_general(
        e_ref[...], w_ref[...],
        (((1,), (1,)), ((), ())),
        preferred_element_type=jnp.float32,
    )


def _fused_table(embed_tokens, lm_head_w):
    return pl.pallas_call(
        _table_body,
        out_shape=jax.ShapeDtypeStruct((_VOCAB, _VOCAB), jnp.float32),
    )(embed_tokens, lm_head_w)


def _make_sc_expand(n_ids, chunk):
    info = plsc.get_sparse_core_info()
    nc, ns = info.num_cores, info.num_subcores
    nw = nc * ns
    per_w = n_ids // nw
    n_chunks = per_w // chunk
    vregs_per_chunk = chunk * _VOCAB // _LANES

    mesh = plsc.VectorSubcoreMesh(core_axis_name="c", subcore_axis_name="s")

    groups_per_chunk = chunk // 4

    @functools.partial(
        pl.kernel,
        mesh=mesh,
        compiler_params=pltpu.CompilerParams(needs_layout_passes=False),
        out_type=jax.ShapeDtypeStruct((n_ids * _VOCAB,), jnp.float32),
        scratch_types=[
            pltpu.VMEM((_VOCAB * _VOCAB,), jnp.float32),
            pltpu.VMEM((chunk,), jnp.int32),
            pltpu.VMEM((chunk * _VOCAB,), jnp.float32),
        ],
    )
    def sc_expand(t_hbm, ids_hbm, out_hbm, t_v, ids_v, out_v):
        wid = lax.axis_index("s") * nc + lax.axis_index("c")
        pltpu.sync_copy(t_hbm, t_v)

        def chunk_body(c, carry):
            base = wid * per_w + c * chunk
            pltpu.sync_copy(ids_hbm.at[pl.ds(base, chunk)], ids_v)

            def group_body(g, carry2):
                lane = lax.iota(jnp.int32, _LANES)
                g4 = g * 4
                for mm in range(5):
                    p = lane + (16 * mm)
                    sel = ((p >= 20).astype(jnp.int32)
                           + (p >= 40).astype(jnp.int32)
                           + (p >= 60).astype(jnp.int32))
                    col = p - sel * _VOCAB
                    iv = sel + g4
                    idv = plsc.load_gather(ids_v, [iv])
                    tix = idv * _VOCAB + col
                    out_v[pl.ds(g * 80 + mm * _LANES, _LANES)] = (
                        plsc.load_gather(t_v, [tix]))
                return carry2

            lax.fori_loop(0, groups_per_chunk, group_body, 0)
            pltpu.sync_copy(
                out_v, out_hbm.at[pl.ds(base * _VOCAB, chunk * _VOCAB)])
            return carry

        lax.fori_loop(0, n_chunks, chunk_body, 0)

    return sc_expand


def kernel(input_ids, embed_tokens, lm_head_w):
    b, l = input_ids.shape
    n_ids = b * l
    table = _fused_table(embed_tokens, lm_head_w)
    ids_flat = input_ids.reshape(n_ids).astype(jnp.int32)
    expand = _make_sc_expand(n_ids, chunk=2048)
    out_flat = expand(table.reshape(_VOCAB * _VOCAB), ids_flat)
    return out_flat.reshape(b, l, _VOCAB)


# speedup vs baseline: 14.7759x; 6.7785x over previous
import functools

import jax
import jax.numpy as jnp
import numpy as np
from jax import lax
from jax.experimental import pallas as pl
from jax.experimental.pallas import tpu as pltpu
from jax.experimental.pallas import tpu_sc as plsc

_VOCAB = 20
_D = 8
_LANES = 16
_TILE_L = 8
_TILE_B = 128


def _table_body(e_ref, w_ref, t_ref):
    t_ref[...] = lax.dot_general(
        e_ref[...], w_ref[...],
        (((1,), (1,)), ((), ())),
        preferred_element_type=jnp.float32,
    )


def _fused_table(embed_tokens, lm_head_w):
    return pl.pallas_call(
        _table_body,
        out_shape=jax.ShapeDtypeStruct((_VOCAB, _VOCAB), jnp.float32),
    )(embed_tokens, lm_head_w)


def _make_sc_expand(n_b, n_l):
    info = plsc.get_sparse_core_info()
    nc, ns = info.num_cores, info.num_subcores
    nw = nc * ns
    b_per_w = n_b // nw
    n_rows = n_l // _TILE_L
    mesh = plsc.VectorSubcoreMesh(core_axis_name="c", subcore_axis_name="s")

    @functools.partial(
        pl.kernel,
        mesh=mesh,
        compiler_params=pltpu.CompilerParams(
            needs_layout_passes=False, use_tc_tiling_on_sc=True),
        out_type=jax.ShapeDtypeStruct((_VOCAB, n_l, n_b), jnp.float32),
        scratch_types=[
            pltpu.VMEM((_VOCAB * _VOCAB,), jnp.float32),
            pltpu.VMEM((_TILE_L, b_per_w), jnp.int32),
            pltpu.VMEM((_VOCAB, _TILE_L, b_per_w), jnp.float32),
        ],
    )
    def sc_expand(t_hbm, ids_hbm, out_hbm, t_v, ids_v, out_v):
        wid = lax.axis_index("s") * nc + lax.axis_index("c")
        b0 = wid * b_per_w
        pltpu.sync_copy(t_hbm, t_v)

        def row_body(r, carry):
            l0 = r * _TILE_L
            pltpu.sync_copy(
                ids_hbm.at[pl.ds(l0, _TILE_L), pl.ds(b0, b_per_w)], ids_v)

            lane = lax.iota(jnp.int32, _LANES)
            for ll in range(_TILE_L):
                def vec_body(k, carry2, ll=ll):
                    idx_b = lane + k * _LANES
                    ids16 = plsc.load_gather(
                        ids_v, [jnp.full((_LANES,), ll, jnp.int32), idx_b])
                    tix0 = ids16 * _VOCAB
                    for v in range(_VOCAB):
                        out_v[v, ll, pl.ds(k * _LANES, _LANES)] = (
                            plsc.load_gather(t_v, [tix0 + v]))
                    return carry2
                lax.fori_loop(0, b_per_w // _LANES, vec_body, 0)

            for v in range(_VOCAB):
                pltpu.sync_copy(
                    out_v.at[v],
                    out_hbm.at[v, pl.ds(l0, _TILE_L), pl.ds(b0, b_per_w)])
            return carry

        lax.fori_loop(0, n_rows, row_body, 0)

    return sc_expand


def kernel(input_ids, embed_tokens, lm_head_w):
    b, l = input_ids.shape
    table = _fused_table(embed_tokens, lm_head_w)
    ids_t = jnp.transpose(input_ids.astype(jnp.int32), (1, 0))
    expand = _make_sc_expand(b, l)
    out_t = expand(table.reshape(_VOCAB * _VOCAB), ids_t)
    return jnp.transpose(out_t, (2, 1, 0))


# speedup vs baseline: 20.1089x; 1.3609x over previous
import functools

import jax
import jax.numpy as jnp
import numpy as np
from jax import lax
from jax.experimental import pallas as pl
from jax.experimental.pallas import tpu as pltpu
from jax.experimental.pallas import tpu_sc as plsc

_VOCAB = 20
_VPAD = 32
_D = 8
_LANES = 16
_TILE_L = 8
_TILE_B = 128


def _table_body(w_ref, e_ref, t_ref):
    t_ref[...] = lax.dot_general(
        w_ref[...], e_ref[...],
        (((1,), (1,)), ((), ())),
        preferred_element_type=jnp.float32,
    )


def _fused_table_t(embed_tokens, lm_head_w):
    e_pad = jnp.zeros((_VPAD, _D), jnp.float32).at[:_VOCAB].set(embed_tokens)
    return pl.pallas_call(
        _table_body,
        out_shape=jax.ShapeDtypeStruct((_VOCAB, _VPAD), jnp.float32),
    )(lm_head_w, e_pad)


def _make_sc_expand(n_b, n_l):
    info = plsc.get_sparse_core_info()
    nc, ns = info.num_cores, info.num_subcores
    nw = nc * ns
    b_per_w = n_b // nw
    n_rows = n_l // _TILE_L
    mesh = plsc.VectorSubcoreMesh(core_axis_name="c", subcore_axis_name="s")

    @functools.partial(
        pl.kernel,
        mesh=mesh,
        compiler_params=pltpu.CompilerParams(
            needs_layout_passes=False, use_tc_tiling_on_sc=True),
        out_type=jax.ShapeDtypeStruct((_VOCAB, n_l, n_b), jnp.float32),
        scratch_types=[
            pltpu.VMEM((_VOCAB * _VPAD,), jnp.float32),
            pltpu.VMEM((_TILE_L, b_per_w), jnp.int32),
            pltpu.VMEM((_VOCAB, _TILE_L, b_per_w), jnp.float32),
        ],
    )
    def sc_expand(t_hbm, ids_hbm, out_hbm, t_v, ids_v, out_v):
        wid = lax.axis_index("s") * nc + lax.axis_index("c")
        b0 = wid * b_per_w
        pltpu.sync_copy(t_hbm, t_v)

        def row_body(r, carry):
            l0 = r * _TILE_L
            pltpu.sync_copy(
                ids_hbm.at[pl.ds(l0, _TILE_L), pl.ds(b0, b_per_w)], ids_v)

            for ll in range(_TILE_L):
                def vec_body(k, carry2, ll=ll):
                    ids16 = ids_v[ll, pl.ds(k * _LANES, _LANES)]
                    for v in range(_VOCAB):
                        out_v[v, ll, pl.ds(k * _LANES, _LANES)] = (
                            plsc.load_gather(
                                t_v.at[pl.ds(v * _VPAD, _VPAD)], [ids16]))
                    return carry2
                lax.fori_loop(0, b_per_w // _LANES, vec_body, 0)

            pltpu.sync_copy(
                out_v,
                out_hbm.at[pl.ds(0, _VOCAB), pl.ds(l0, _TILE_L),
                           pl.ds(b0, b_per_w)])
            return carry

        lax.fori_loop(0, n_rows, row_body, 0)

    return sc_expand


def kernel(input_ids, embed_tokens, lm_head_w):
    b, l = input_ids.shape
    table_t = _fused_table_t(embed_tokens, lm_head_w)
    ids_t = jnp.transpose(input_ids.astype(jnp.int32), (1, 0))
    expand = _make_sc_expand(b, l)
    out_t = expand(table_t.reshape(_VOCAB * _VPAD), ids_t)
    return jnp.transpose(out_t, (2, 1, 0))


# speedup vs baseline: 22.6491x; 1.1263x over previous
import functools

import jax
import jax.numpy as jnp
import numpy as np
from jax import lax
from jax.experimental import pallas as pl
from jax.experimental.pallas import tpu as pltpu
from jax.experimental.pallas import tpu_sc as plsc

_VOCAB = 20
_VPAD = 32
_D = 8
_LANES = 16
_TILE_L = 8
_TILE_B = 128


def _table_body(w_ref, e_ref, t_ref):
    t_ref[...] = lax.dot_general(
        w_ref[...], e_ref[...],
        (((1,), (1,)), ((), ())),
        preferred_element_type=jnp.float32,
    )


def _fused_table_t(embed_tokens, lm_head_w):
    e_pad = jnp.zeros((_VPAD, _D), jnp.float32).at[:_VOCAB].set(embed_tokens)
    return pl.pallas_call(
        _table_body,
        out_shape=jax.ShapeDtypeStruct((_VOCAB, _VPAD), jnp.float32),
    )(lm_head_w, e_pad)


def _make_sc_expand(n_b, n_l):
    info = plsc.get_sparse_core_info()
    nc, ns = info.num_cores, info.num_subcores
    nw = nc * ns
    b_per_w = n_b // nw
    n_rows = n_l // _TILE_L
    mesh = plsc.VectorSubcoreMesh(core_axis_name="c", subcore_axis_name="s")

    b_half = b_per_w // 2

    @functools.partial(
        pl.kernel,
        mesh=mesh,
        compiler_params=pltpu.CompilerParams(
            needs_layout_passes=False, use_tc_tiling_on_sc=True),
        out_type=jax.ShapeDtypeStruct((_VOCAB, n_l, n_b), jnp.float32),
        scratch_types=[
            pltpu.VMEM((_VOCAB * _VPAD,), jnp.float32),
            pltpu.VMEM((_TILE_L, b_per_w), jnp.int32),
            pltpu.VMEM((_VOCAB, _TILE_L, b_half), jnp.float32),
            pltpu.VMEM((_VOCAB, _TILE_L, b_half), jnp.float32),
            pltpu.SemaphoreType.DMA,
            pltpu.SemaphoreType.DMA,
        ],
    )
    def sc_expand(t_hbm, ids_hbm, out_hbm, t_v, ids_v, out_v0, out_v1,
                  sem0, sem1):
        wid = lax.axis_index("s") * nc + lax.axis_index("c")
        b0 = wid * b_per_w
        pltpu.sync_copy(t_hbm, t_v)

        def row_body(r, carry):
            l0 = r * _TILE_L
            pltpu.sync_copy(
                ids_hbm.at[pl.ds(l0, _TILE_L), pl.ds(b0, b_per_w)], ids_v)

            for h, (out_v, sem) in enumerate(((out_v0, sem0),
                                              (out_v1, sem1))):
                dst = out_hbm.at[pl.ds(0, _VOCAB), pl.ds(l0, _TILE_L),
                                 pl.ds(b0 + h * b_half, b_half)]

                @pl.when(r > 0)
                def _(out_v=out_v, sem=sem, dst=dst):
                    pltpu.make_async_copy(out_v, dst, sem).wait()

                for ll in range(_TILE_L):
                    def vec_body(k, carry2, ll=ll, h=h, out_v=out_v):
                        ids16 = ids_v[ll, pl.ds(h * b_half + k * _LANES,
                                                _LANES)]
                        for v in range(_VOCAB):
                            out_v[v, ll, pl.ds(k * _LANES, _LANES)] = (
                                plsc.load_gather(
                                    t_v.at[pl.ds(v * _VPAD, _VPAD)],
                                    [ids16]))
                        return carry2
                    lax.fori_loop(0, b_half // _LANES, vec_body, 0)

                pltpu.async_copy(out_v, dst, sem)
            return carry

        lax.fori_loop(0, n_rows, row_body, 0)

        l_last = (n_rows - 1) * _TILE_L
        for h, (out_v, sem) in enumerate(((out_v0, sem0), (out_v1, sem1))):
            dst = out_hbm.at[pl.ds(0, _VOCAB), pl.ds(l_last, _TILE_L),
                             pl.ds(b0 + h * b_half, b_half)]
            pltpu.make_async_copy(out_v, dst, sem).wait()

    return sc_expand


def kernel(input_ids, embed_tokens, lm_head_w):
    b, l = input_ids.shape
    table_t = _fused_table_t(embed_tokens, lm_head_w)
    ids_t = jnp.transpose(input_ids.astype(jnp.int32), (1, 0))
    expand = _make_sc_expand(b, l)
    out_t = expand(table_t.reshape(_VOCAB * _VPAD), ids_t)
    return jnp.transpose(out_t, (2, 1, 0))


# speedup vs baseline: 22.8510x; 1.0089x over previous
import functools

import jax
import jax.numpy as jnp
import numpy as np
from jax import lax
from jax.experimental import pallas as pl
from jax.experimental.pallas import tpu as pltpu
from jax.experimental.pallas import tpu_sc as plsc

_VOCAB = 20
_VPAD = 32
_D = 8
_LANES = 16
_TILE_L = 8
_TILE_B = 128


def _table_body(w_ref, e_ref, t_ref):
    t_ref[...] = lax.dot_general(
        w_ref[...], e_ref[...],
        (((1,), (1,)), ((), ())),
        preferred_element_type=jnp.float32,
    )


def _fused_table_t(embed_tokens, lm_head_w):
    e_pad = jnp.zeros((_VPAD, _D), jnp.float32).at[:_VOCAB].set(embed_tokens)
    return pl.pallas_call(
        _table_body,
        out_shape=jax.ShapeDtypeStruct((_VOCAB, _VPAD), jnp.float32),
    )(lm_head_w, e_pad)


def _make_sc_expand(n_b, n_l):
    info = plsc.get_sparse_core_info()
    nc, ns = info.num_cores, info.num_subcores
    nw = nc * ns
    b_per_w = n_b // nw
    n_rows = n_l // _TILE_L
    mesh = plsc.VectorSubcoreMesh(core_axis_name="c", subcore_axis_name="s")

    b_half = b_per_w // 2

    @functools.partial(
        pl.kernel,
        mesh=mesh,
        compiler_params=pltpu.CompilerParams(
            needs_layout_passes=False, use_tc_tiling_on_sc=True),
        out_type=jax.ShapeDtypeStruct((_VOCAB, n_l, n_b), jnp.float32),
        scratch_types=[
            pltpu.VMEM((_VOCAB * _VPAD,), jnp.float32),
            pltpu.VMEM((2 * _TILE_L, b_per_w), jnp.int32),
            pltpu.VMEM((_VOCAB, _TILE_L, b_half), jnp.float32),
            pltpu.VMEM((_VOCAB, _TILE_L, b_half), jnp.float32),
            pltpu.SemaphoreType.DMA,
            pltpu.SemaphoreType.DMA,
            pltpu.SemaphoreType.DMA,
        ],
    )
    def sc_expand(t_hbm, ids_hbm, out_hbm, t_v, ids_v, out_v0, out_v1,
                  sem0, sem1, isem):
        wid = lax.axis_index("s") * nc + lax.axis_index("c")
        b0 = wid * b_per_w
        pltpu.sync_copy(t_hbm, t_v)

        def ids_src(r):
            return ids_hbm.at[pl.ds(r * _TILE_L, _TILE_L),
                              pl.ds(b0, b_per_w)]

        pltpu.async_copy(ids_src(0), ids_v.at[pl.ds(0, _TILE_L)], isem)

        def row_body(r, carry):
            l0 = r * _TILE_L
            par8 = (r % 2) * _TILE_L
            pltpu.make_async_copy(
                ids_src(r), ids_v.at[pl.ds(par8, _TILE_L)], isem).wait()

            @pl.when(r < n_rows - 1)
            def _():
                pltpu.async_copy(
                    ids_src(r + 1),
                    ids_v.at[pl.ds(_TILE_L - par8, _TILE_L)], isem)

            for h, (out_v, sem) in enumerate(((out_v0, sem0),
                                              (out_v1, sem1))):
                dst = out_hbm.at[pl.ds(0, _VOCAB), pl.ds(l0, _TILE_L),
                                 pl.ds(b0 + h * b_half, b_half)]

                @pl.when(r > 0)
                def _(out_v=out_v, sem=sem, dst=dst):
                    pltpu.make_async_copy(out_v, dst, sem).wait()

                for ll in range(_TILE_L):
                    def vec_body(k, carry2, ll=ll, h=h, out_v=out_v):
                        ids16 = ids_v[par8 + ll,
                                      pl.ds(h * b_half + k * _LANES,
                                            _LANES)]
                        for v in range(_VOCAB):
                            out_v[v, ll, pl.ds(k * _LANES, _LANES)] = (
                                plsc.load_gather(
                                    t_v.at[pl.ds(v * _VPAD, _VPAD)],
                                    [ids16]))
                        return carry2
                    lax.fori_loop(0, b_half // _LANES, vec_body, 0)

                pltpu.async_copy(out_v, dst, sem)
            return carry

        lax.fori_loop(0, n_rows, row_body, 0)

        l_last = (n_rows - 1) * _TILE_L
        for h, (out_v, sem) in enumerate(((out_v0, sem0), (out_v1, sem1))):
            dst = out_hbm.at[pl.ds(0, _VOCAB), pl.ds(l_last, _TILE_L),
                             pl.ds(b0 + h * b_half, b_half)]
            pltpu.make_async_copy(out_v, dst, sem).wait()

    return sc_expand


def kernel(input_ids, embed_tokens, lm_head_w):
    b, l = input_ids.shape
    table_t = _fused_table_t(embed_tokens, lm_head_w)
    ids_t = jnp.transpose(input_ids.astype(jnp.int32), (1, 0))
    expand = _make_sc_expand(b, l)
    out_t = expand(table_t.reshape(_VOCAB * _VPAD), ids_t)
    return jnp.transpose(out_t, (2, 1, 0))


# speedup vs baseline: 64.6873x; 2.8308x over previous
import functools

import jax
import jax.numpy as jnp
import numpy as np
from jax import lax
from jax.experimental import pallas as pl
from jax.experimental.pallas import tpu as pltpu
from jax.experimental.pallas import tpu_sc as plsc

_VOCAB = 20
_VPAD = 32
_D = 8
_LANES = 16
_TILE_L = 8
_TILE_B = 128


def _table_body(w_ref, e_ref, t_ref):
    t_ref[...] = lax.dot_general(
        w_ref[...], e_ref[...],
        (((1,), (1,)), ((), ())),
        preferred_element_type=jnp.float32,
    )


def _fused_table_t(embed_tokens, lm_head_w):
    e_pad = jnp.zeros((_VPAD, _D), jnp.float32).at[:_VOCAB].set(embed_tokens)
    return pl.pallas_call(
        _table_body,
        out_shape=jax.ShapeDtypeStruct((_VOCAB, _VPAD), jnp.float32),
    )(lm_head_w, e_pad)


def _make_sc_expand(n_b, n_l):
    info = plsc.get_sparse_core_info()
    nc, ns = info.num_cores, info.num_subcores
    nw = nc * ns
    b_per_w = n_b // nw
    n_rows = n_l // _TILE_L
    mesh = plsc.VectorSubcoreMesh(core_axis_name="c", subcore_axis_name="s")

    b_half = b_per_w // 2

    @functools.partial(
        pl.kernel,
        mesh=mesh,
        compiler_params=pltpu.CompilerParams(
            needs_layout_passes=False, use_tc_tiling_on_sc=True),
        out_type=jax.ShapeDtypeStruct((_VOCAB, n_l, n_b), jnp.float32),
        scratch_types=[
            pltpu.VMEM((_VOCAB * _VPAD,), jnp.float32),
            pltpu.VMEM((2 * _TILE_L, b_per_w), jnp.int32),
            pltpu.VMEM((_VOCAB, _TILE_L, b_half), jnp.float32),
            pltpu.VMEM((_VOCAB, _TILE_L, b_half), jnp.float32),
            pltpu.SemaphoreType.DMA,
            pltpu.SemaphoreType.DMA,
            pltpu.SemaphoreType.DMA,
        ],
    )
    def sc_expand(t_hbm, ids_hbm, out_hbm, t_v, ids_v, out_v0, out_v1,
                  sem0, sem1, isem):
        wid = lax.axis_index("s") * nc + lax.axis_index("c")
        b0 = wid * b_per_w
        pltpu.sync_copy(t_hbm, t_v)

        def ids_src(r):
            return ids_hbm.at[pl.ds(r * _TILE_L, _TILE_L),
                              pl.ds(b0, b_per_w)]

        pltpu.async_copy(ids_src(0), ids_v.at[pl.ds(0, _TILE_L)], isem)

        def row_body(r, carry):
            l0 = r * _TILE_L
            par8 = (r % 2) * _TILE_L
            pltpu.make_async_copy(
                ids_src(r), ids_v.at[pl.ds(par8, _TILE_L)], isem).wait()

            @pl.when(r < n_rows - 1)
            def _():
                pltpu.async_copy(
                    ids_src(r + 1),
                    ids_v.at[pl.ds(_TILE_L - par8, _TILE_L)], isem)

            for h, (out_v, sem) in enumerate(((out_v0, sem0),
                                              (out_v1, sem1))):
                dst = out_hbm.at[pl.ds(0, _VOCAB), pl.ds(l0, _TILE_L),
                                 pl.ds(b0 + h * b_half, b_half)]

                @pl.when(r > 0)
                def _(out_v=out_v, sem=sem, dst=dst):
                    pltpu.make_async_copy(out_v, dst, sem).wait()

                for ll in range(_TILE_L):
                    def vec_body(k, carry2, ll=ll, h=h, out_v=out_v):
                        ids16 = ids_v[par8 + ll,
                                      pl.ds(h * b_half + k * _LANES,
                                            _LANES)]
                        rows = [
                            plsc.load_gather(
                                t_v.at[pl.ds(v * _VPAD, _VPAD)], [ids16])
                            for v in range(_VOCAB)
                        ]
                        for v in range(_VOCAB):
                            out_v[v, ll, pl.ds(k * _LANES, _LANES)] = (
                                rows[v])
                        return carry2
                    lax.fori_loop(0, b_half // _LANES, vec_body, 0)

                pltpu.async_copy(out_v, dst, sem)
            return carry

        lax.fori_loop(0, n_rows, row_body, 0)

        l_last = (n_rows - 1) * _TILE_L
        for h, (out_v, sem) in enumerate(((out_v0, sem0), (out_v1, sem1))):
            dst = out_hbm.at[pl.ds(0, _VOCAB), pl.ds(l_last, _TILE_L),
                             pl.ds(b0 + h * b_half, b_half)]
            pltpu.make_async_copy(out_v, dst, sem).wait()

    return sc_expand


def kernel(input_ids, embed_tokens, lm_head_w):
    b, l = input_ids.shape
    table_t = _fused_table_t(embed_tokens, lm_head_w)
    ids_t = jnp.transpose(input_ids.astype(jnp.int32), (1, 0))
    expand = _make_sc_expand(b, l)
    out_t = expand(table_t.reshape(_VOCAB * _VPAD), ids_t)
    return jnp.transpose(out_t, (2, 1, 0))


# speedup vs baseline: 65.0453x; 1.0055x over previous
import functools

import jax
import jax.numpy as jnp
import numpy as np
from jax import lax
from jax.experimental import pallas as pl
from jax.experimental.pallas import tpu as pltpu
from jax.experimental.pallas import tpu_sc as plsc

_VOCAB = 20
_VPAD = 32
_D = 8
_LANES = 16
_TILE_L = 8
_TILE_B = 128


def _table_body(w_ref, e_ref, t_ref):
    t_ref[...] = lax.dot_general(
        w_ref[...], e_ref[...],
        (((1,), (1,)), ((), ())),
        preferred_element_type=jnp.float32,
    )


def _fused_table_t(embed_tokens, lm_head_w):
    e_pad = jnp.zeros((_VPAD, _D), jnp.float32).at[:_VOCAB].set(embed_tokens)
    return pl.pallas_call(
        _table_body,
        out_shape=jax.ShapeDtypeStruct((_VOCAB, _VPAD), jnp.float32),
    )(lm_head_w, e_pad)


def _make_sc_expand(n_b, n_l):
    info = plsc.get_sparse_core_info()
    nc, ns = info.num_cores, info.num_subcores
    nw = nc * ns
    b_per_w = n_b // nw
    n_rows = n_l // _TILE_L
    mesh = plsc.VectorSubcoreMesh(core_axis_name="c", subcore_axis_name="s")

    b_half = b_per_w // 2

    @functools.partial(
        pl.kernel,
        mesh=mesh,
        compiler_params=pltpu.CompilerParams(
            needs_layout_passes=False, use_tc_tiling_on_sc=True),
        out_type=jax.ShapeDtypeStruct((_VOCAB, n_l, n_b), jnp.float32),
        scratch_types=[
            pltpu.VMEM((_VOCAB * _VPAD,), jnp.float32),
            pltpu.VMEM((2 * _TILE_L, b_per_w), jnp.int32),
            pltpu.VMEM((_VOCAB, _TILE_L, b_half), jnp.float32),
            pltpu.VMEM((_VOCAB, _TILE_L, b_half), jnp.float32),
            pltpu.SemaphoreType.DMA,
            pltpu.SemaphoreType.DMA,
            pltpu.SemaphoreType.DMA,
        ],
    )
    def sc_expand(t_hbm, ids_hbm, out_hbm, t_v, ids_v, out_v0, out_v1,
                  sem0, sem1, isem):
        wid = lax.axis_index("s") * nc + lax.axis_index("c")
        b0 = wid * b_per_w
        pltpu.sync_copy(t_hbm, t_v)

        def ids_src(r):
            return ids_hbm.at[pl.ds(r * _TILE_L, _TILE_L),
                              pl.ds(b0, b_per_w)]

        pltpu.async_copy(ids_src(0), ids_v.at[pl.ds(0, _TILE_L)], isem)

        def row_body(r, carry):
            l0 = r * _TILE_L
            par8 = (r % 2) * _TILE_L
            pltpu.make_async_copy(
                ids_src(r), ids_v.at[pl.ds(par8, _TILE_L)], isem).wait()

            @pl.when(r < n_rows - 1)
            def _():
                pltpu.async_copy(
                    ids_src(r + 1),
                    ids_v.at[pl.ds(_TILE_L - par8, _TILE_L)], isem)

            for h, (out_v, sem) in enumerate(((out_v0, sem0),
                                              (out_v1, sem1))):
                dst = out_hbm.at[pl.ds(0, _VOCAB), pl.ds(l0, _TILE_L),
                                 pl.ds(b0 + h * b_half, b_half)]

                @pl.when(r > 0)
                def _(out_v=out_v, sem=sem, dst=dst):
                    pltpu.make_async_copy(out_v, dst, sem).wait()

                for ll in range(_TILE_L):
                    @plsc.parallel_loop(0, b_half // _LANES, unroll=2)
                    def vec_body(k, ll=ll, h=h, out_v=out_v):
                        ids16 = ids_v[par8 + ll,
                                      pl.ds(h * b_half + k * _LANES,
                                            _LANES)]
                        rows = [
                            plsc.load_gather(
                                t_v.at[pl.ds(v * _VPAD, _VPAD)], [ids16])
                            for v in range(_VOCAB)
                        ]
                        for v in range(_VOCAB):
                            out_v[v, ll, pl.ds(k * _LANES, _LANES)] = (
                                rows[v])

                pltpu.async_copy(out_v, dst, sem)
            return carry

        lax.fori_loop(0, n_rows, row_body, 0)

        l_last = (n_rows - 1) * _TILE_L
        for h, (out_v, sem) in enumerate(((out_v0, sem0), (out_v1, sem1))):
            dst = out_hbm.at[pl.ds(0, _VOCAB), pl.ds(l_last, _TILE_L),
                             pl.ds(b0 + h * b_half, b_half)]
            pltpu.make_async_copy(out_v, dst, sem).wait()

    return sc_expand


def kernel(input_ids, embed_tokens, lm_head_w):
    b, l = input_ids.shape
    table_t = _fused_table_t(embed_tokens, lm_head_w)
    ids_t = jnp.transpose(input_ids.astype(jnp.int32), (1, 0))
    expand = _make_sc_expand(b, l)
    out_t = expand(table_t.reshape(_VOCAB * _VPAD), ids_t)
    return jnp.transpose(out_t, (2, 1, 0))
